# baseline (speedup 1.0000x reference)
import functools

import jax
import jax.numpy as jnp
from jax import lax
from jax.experimental import pallas as pl
from jax.experimental.pallas import tpu as pltpu
from jax.experimental.pallas import tpu_sc as plsc

N = 10000
R = 4
E = 80000
IN_FEATS = 128
HID = 32
OUT = 2

NTILES = 16
EPT = E // NTILES
CHUNK = 1000

_MESH = plsc.VectorSubcoreMesh(core_axis_name="c", subcore_axis_name="s")


@functools.partial(
    pl.kernel,
    mesh=_MESH,
    compiler_params=pltpu.CompilerParams(use_tc_tiling_on_sc=False),
    out_type=jax.ShapeDtypeStruct((R * 2 * N,), jnp.float32),
    scratch_types=[
        pltpu.VMEM_SHARED((N,), jnp.float32),
        pltpu.VMEM_SHARED((N,), jnp.float32),
        pltpu.VMEM_SHARED((N,), jnp.float32),
        pltpu.VMEM_SHARED((N,), jnp.float32),
        pltpu.VMEM((EPT,), jnp.int32),
        pltpu.VMEM((EPT,), jnp.float32),
        pltpu.VMEM((N,), jnp.float32),
    ],
)
def _deg_call(s0, d0, s1, d1, s2, d2, s3, d3, zeros_n, ones_e,
              deg_out, tA, tB, tC, tD, idx_v, ones_v, bounce_v):
    c = lax.axis_index("c")
    s = lax.axis_index("s")
    pltpu.sync_copy(ones_e, ones_v)

    @pl.when(s < 4)
    def _():
        pltpu.sync_copy(zeros_n, bounce_v)

    @pl.when(s == 0)
    def _():
        pltpu.sync_copy(bounce_v, tA)

    @pl.when(s == 1)
    def _():
        pltpu.sync_copy(bounce_v, tB)

    @pl.when(s == 2)
    def _():
        pltpu.sync_copy(bounce_v, tC)

    @pl.when(s == 3)
    def _():
        pltpu.sync_copy(bounce_v, tD)

    plsc.subcore_barrier()

    def scat(src_ref, tab):
        pltpu.sync_copy(src_ref.at[pl.ds(s * EPT, EPT)], idx_v)
        pltpu.sync_copy(ones_v, tab.at[idx_v], add=True)

    @pl.when(c == 0)
    def _():
        scat(s0, tA)
        scat(d0, tB)
        scat(s1, tC)
        scat(d1, tD)

    @pl.when(c == 1)
    def _():
        scat(s2, tA)
        scat(d2, tB)
        scat(s3, tC)
        scat(d3, tD)

    plsc.subcore_barrier()

    @pl.when(s == 0)
    def _():
        pltpu.sync_copy(tA, bounce_v)
        pltpu.sync_copy(bounce_v, deg_out.at[pl.ds((c * 4 + 0) * N, N)])

    @pl.when(s == 1)
    def _():
        pltpu.sync_copy(tB, bounce_v)
        pltpu.sync_copy(bounce_v, deg_out.at[pl.ds((c * 4 + 1) * N, N)])

    @pl.when(s == 2)
    def _():
        pltpu.sync_copy(tC, bounce_v)
        pltpu.sync_copy(bounce_v, deg_out.at[pl.ds((c * 4 + 2) * N, N)])

    @pl.when(s == 3)
    def _():
        pltpu.sync_copy(tD, bounce_v)
        pltpu.sync_copy(bounce_v, deg_out.at[pl.ds((c * 4 + 3) * N, N)])


_BN = 2000
_NB = N // _BN


def _proj_body(xp_ref, bd_ref, scale_ref, out_ref):
    xp = xp_ref[...]
    for r in range(R):
        y = jnp.dot(xp, bd_ref[r], preferred_element_type=jnp.float32)
        out_ref[r] = y * scale_ref[r]


def _proj_call(xp, bd, scale_p):
    return pl.pallas_call(
        _proj_body,
        out_shape=jax.ShapeDtypeStruct((R, N // 4, 128), jnp.float32),
    )(xp, bd, scale_p)


@functools.partial(
    pl.kernel,
    mesh=_MESH,
    compiler_params=pltpu.CompilerParams(use_tc_tiling_on_sc=False),
    out_type=(
        jax.ShapeDtypeStruct((R * N, HID), jnp.float32),
        jax.ShapeDtypeStruct((R * N,), jnp.float32),
    ),
    scratch_types=[
        pltpu.VMEM_SHARED((N, HID), jnp.float32),
        pltpu.VMEM_SHARED((N, HID), jnp.float32),
        pltpu.VMEM_SHARED((N,), jnp.float32),
        pltpu.VMEM_SHARED((N,), jnp.float32),
        pltpu.VMEM((CHUNK,), jnp.int32),
        pltpu.VMEM((CHUNK,), jnp.int32),
        pltpu.VMEM((CHUNK,), jnp.int32),
        pltpu.VMEM((CHUNK,), jnp.int32),
        pltpu.VMEM((CHUNK, HID), jnp.float32),
        pltpu.VMEM((CHUNK, HID), jnp.float32),
        pltpu.VMEM((CHUNK,), jnp.float32),
        pltpu.VMEM((CHUNK,), jnp.float32),
        pltpu.VMEM((N,), jnp.float32),
        pltpu.SemaphoreType.DMA,
        pltpu.SemaphoreType.DMA,
        pltpu.SemaphoreType.DMA,
        pltpu.SemaphoreType.DMA,
        pltpu.SemaphoreType.DMA,
        pltpu.SemaphoreType.DMA,
    ],
)
def _edge_call(s0, d0, s1, d1, s2, d2, s3, d3,
               yn0, yn1, yn2, yn3, rin0, rin1, rin2, rin3,
               zeros_2d, zeros_n,
               agg_out, s_out,
               aggA, aggB, sA, sB,
               isrcA, isrcB, idstA, idstB, rowsA, rowsB, wA, wB,
               sb_v,
               semG, semG2, semSA, semSB, semWA, semWB):
    c = lax.axis_index("c")
    s = lax.axis_index("s")
    NCH = N // CHUNK
    NCHK = EPT // CHUNK

    isrc_b = (isrcA, isrcB)
    idst_b = (idstA, idstB)
    rows_b = (rowsA, rowsB)
    w_b = (wA, wB)
    semS_b = (semSA, semSB)
    semW_b = (semWA, semWB)

    @pl.when(s < 2)
    def _():
        pltpu.sync_copy(zeros_2d, rowsA)

    @pl.when((s == 2) | (s == 3))
    def _():
        pltpu.sync_copy(zeros_n, sb_v)

    @pl.when(s == 0)
    def _():
        for k in range(NCH):
            pltpu.sync_copy(rowsA, aggA.at[pl.ds(k * CHUNK, CHUNK)])

    @pl.when(s == 1)
    def _():
        for k in range(NCH):
            pltpu.sync_copy(rowsA, aggB.at[pl.ds(k * CHUNK, CHUNK)])

    @pl.when(s == 2)
    def _():
        pltpu.sync_copy(sb_v, sA)

    @pl.when(s == 3)
    def _():
        pltpu.sync_copy(sb_v, sB)

    plsc.subcore_barrier()

    def proc(src_ref, dst_ref, yn_ref, rin_ref, agg_sh, s_sh):
        pending = []
        for k in range(NCHK):
            b = k % 2
            if k >= 2:
                pending[k - 2][0].wait()
                pending[k - 2][1].wait()
            base = s * EPT + k * CHUNK
            pltpu.sync_copy(src_ref.at[pl.ds(base, CHUNK)], isrc_b[b])
            pltpu.sync_copy(dst_ref.at[pl.ds(base, CHUNK)], idst_b[b])
            gd = pltpu.async_copy(yn_ref.at[isrc_b[b]], rows_b[b], semG)
            wd = pltpu.async_copy(rin_ref.at[idst_b[b]], w_b[b], semG2)
            gd.wait()
            wd.wait()
            sc_rows = pltpu.async_copy(rows_b[b], agg_sh.at[idst_b[b]],
                                       semS_b[b], add=True)
            sc_w = pltpu.async_copy(w_b[b], s_sh.at[isrc_b[b]],
                                    semW_b[b], add=True)
            pending.append((sc_rows, sc_w))
        for k in range(max(0, NCHK - 2), NCHK):
            pending[k][0].wait()
            pending[k][1].wait()

    @pl.when(c == 0)
    def _():
        proc(s0, d0, yn0, rin0, aggA, sA)
        proc(s1, d1, yn1, rin1, aggB, sB)

    @pl.when(c == 1)
    def _():
        proc(s2, d2, yn2, rin2, aggA, sA)
        proc(s3, d3, yn3, rin3, aggB, sB)

    plsc.subcore_barrier()

    @pl.when(s == 0)
    def _():
        for k in range(NCH):
            pltpu.sync_copy(aggA.at[pl.ds(k * CHUNK, CHUNK)], rowsA)
            pltpu.sync_copy(
                rowsA, agg_out.at[pl.ds((c * 2 + 0) * N + k * CHUNK, CHUNK)])

    @pl.when(s == 1)
    def _():
        for k in range(NCH):
            pltpu.sync_copy(aggB.at[pl.ds(k * CHUNK, CHUNK)], rowsA)
            pltpu.sync_copy(
                rowsA, agg_out.at[pl.ds((c * 2 + 1) * N + k * CHUNK, CHUNK)])

    @pl.when(s == 2)
    def _():
        pltpu.sync_copy(sA, sb_v)
        pltpu.sync_copy(sb_v, s_out.at[pl.ds((c * 2 + 0) * N, N)])

    @pl.when(s == 3)
    def _():
        pltpu.sync_copy(sB, sb_v)
        pltpu.sync_copy(sb_v, s_out.at[pl.ds((c * 2 + 1) * N, N)])


def _pool_body(agg_ref, sp_ref, rin_ref, rout_ref, b1_ref, w2_ref, b2_ref,
               wc_ref, bc_ref, out_ref, v_acc):
    i = pl.program_id(0)

    @pl.when(i == 0)
    def _():
        v_acc[...] = jnp.zeros((R, HID), jnp.float32)

    agg = agg_ref[...]
    hsum = (agg[0] * rin_ref[:, 0:1] + agg[1] * rin_ref[:, 1:2]
            + agg[2] * rin_ref[:, 2:3] + agg[3] * rin_ref[:, 3:4])
    b1s = jnp.sum(b1_ref[...], axis=0, keepdims=True)
    h = jnp.maximum(hsum + b1s, 0.0)
    cmat = rout_ref[...] * sp_ref[...]
    v_acc[...] += lax.dot_general(cmat, h, (((0,), (0,)), ((), ())),
                                  preferred_element_type=jnp.float32)

    @pl.when(i == _NB - 1)
    def _():
        V = v_acc[...]
        vw = lax.dot_general(V, w2_ref[...], (((1,), (1,)), ((0,), (0,))),
                             preferred_element_type=jnp.float32)
        ge = (jnp.sum(vw, axis=0, keepdims=True) * (1.0 / N)
              + jnp.sum(b2_ref[...], axis=0, keepdims=True))
        out_ref[...] = (jnp.dot(ge, wc_ref[...],
                                preferred_element_type=jnp.float32)
                        + bc_ref[...])


def _pool_call(agg, sp_t, rin_t, rout_t, b1, W2, b2, Wc, bc2):
    return pl.pallas_call(
        _pool_body,
        grid=(_NB,),
        in_specs=[
            pl.BlockSpec((R, _BN, HID), lambda i: (0, i, 0)),
            pl.BlockSpec((_BN, R), lambda i: (i, 0)),
            pl.BlockSpec((_BN, R), lambda i: (i, 0)),
            pl.BlockSpec((_BN, R), lambda i: (i, 0)),
            pl.BlockSpec((R, HID), lambda i: (0, 0)),
            pl.BlockSpec((R, HID, HID), lambda i: (0, 0, 0)),
            pl.BlockSpec((R, HID), lambda i: (0, 0)),
            pl.BlockSpec((HID, OUT), lambda i: (0, 0)),
            pl.BlockSpec((1, OUT), lambda i: (0, 0)),
        ],
        out_specs=pl.BlockSpec((1, OUT), lambda i: (0, 0)),
        out_shape=jax.ShapeDtypeStruct((1, OUT), jnp.float32),
        scratch_shapes=[pltpu.VMEM((R, HID), jnp.float32)],
    )(agg, sp_t, rin_t, rout_t, b1, W2, b2, Wc, bc2)


@jax.jit
def kernel(x, edge_index, W1, b1, W2, b2, Wc, bc):
    srcs = [edge_index[r, 0] for r in range(R)]
    dsts = [edge_index[r, 1] for r in range(R)]
    zeros_n = jnp.zeros((N,), jnp.float32)
    zeros_2d = jnp.zeros((CHUNK, HID), jnp.float32)
    ones_e = jnp.ones((EPT,), jnp.float32)

    deg_flat = _deg_call(srcs[0], dsts[0], srcs[1], dsts[1],
                         srcs[2], dsts[2], srcs[3], dsts[3],
                         zeros_n, ones_e)
    deg = deg_flat.reshape(R, 2, N)
    rout = lax.rsqrt(jnp.maximum(deg[:, 0], 1.0))
    rin = lax.rsqrt(jnp.maximum(deg[:, 1], 1.0))
    rout_t = rout.T
    rin_t = rin.T

    xp = x.reshape(N // 4, 4 * IN_FEATS)
    bd = jnp.zeros((R, 4 * IN_FEATS, 4 * HID), jnp.float32)
    for g in range(4):
        bd = bd.at[:, g * IN_FEATS:(g + 1) * IN_FEATS,
                   g * HID:(g + 1) * HID].set(W1)
    scale_p = jnp.repeat(rout_t.reshape(N // 4, 4, R).transpose(2, 0, 1),
                         HID, axis=2)
    yn = _proj_call(xp, bd, scale_p).reshape(R, N, HID)

    agg_f, s_f = _edge_call(
        srcs[0], dsts[0], srcs[1], dsts[1], srcs[2], dsts[2], srcs[3], dsts[3],
        yn[0], yn[1], yn[2], yn[3],
        rin[0], rin[1], rin[2], rin[3],
        zeros_2d, zeros_n)

    logits = _pool_call(agg_f.reshape(R, N, HID), s_f.reshape(R, N).T,
                        rin_t, rout_t, b1, W2, b2, Wc, bc.reshape(1, OUT))
    return logits.reshape(OUT)


# speedup vs baseline: 1.2527x; 1.2527x over previous
import functools

import jax
import jax.numpy as jnp
from jax import lax
from jax.experimental import pallas as pl
from jax.experimental.pallas import tpu as pltpu
from jax.experimental.pallas import tpu_sc as plsc

N = 10000
R = 4
E = 80000
IN_FEATS = 128
HID = 32
OUT = 2

NTILES = 16
EPT = E // NTILES
CHUNK = 1000

_MESH = plsc.VectorSubcoreMesh(core_axis_name="c", subcore_axis_name="s")


@functools.partial(
    pl.kernel,
    mesh=_MESH,
    compiler_params=pltpu.CompilerParams(use_tc_tiling_on_sc=False),
    out_type=jax.ShapeDtypeStruct((R * 2 * N,), jnp.float32),
    scratch_types=[
        pltpu.VMEM_SHARED((N,), jnp.float32),
        pltpu.VMEM_SHARED((N,), jnp.float32),
        pltpu.VMEM_SHARED((N,), jnp.float32),
        pltpu.VMEM_SHARED((N,), jnp.float32),
        pltpu.VMEM((EPT,), jnp.int32),
        pltpu.VMEM((EPT,), jnp.float32),
        pltpu.VMEM((N,), jnp.float32),
    ],
)
def _deg_call(s0, d0, s1, d1, s2, d2, s3, d3, zeros_n, ones_e,
              deg_out, tA, tB, tC, tD, idx_v, ones_v, bounce_v):
    c = lax.axis_index("c")
    s = lax.axis_index("s")
    pltpu.sync_copy(ones_e, ones_v)

    @pl.when(s < 4)
    def _():
        pltpu.sync_copy(zeros_n, bounce_v)

    @pl.when(s == 0)
    def _():
        pltpu.sync_copy(bounce_v, tA)

    @pl.when(s == 1)
    def _():
        pltpu.sync_copy(bounce_v, tB)

    @pl.when(s == 2)
    def _():
        pltpu.sync_copy(bounce_v, tC)

    @pl.when(s == 3)
    def _():
        pltpu.sync_copy(bounce_v, tD)

    plsc.subcore_barrier()

    def scat(src_ref, tab):
        pltpu.sync_copy(src_ref.at[pl.ds(s * EPT, EPT)], idx_v)
        pltpu.sync_copy(ones_v, tab.at[idx_v], add=True)

    @pl.when(c == 0)
    def _():
        scat(s0, tA)
        scat(d0, tB)
        scat(s1, tC)
        scat(d1, tD)

    @pl.when(c == 1)
    def _():
        scat(s2, tA)
        scat(d2, tB)
        scat(s3, tC)
        scat(d3, tD)

    plsc.subcore_barrier()

    @pl.when(s == 0)
    def _():
        pltpu.sync_copy(tA, bounce_v)
        pltpu.sync_copy(bounce_v, deg_out.at[pl.ds((c * 4 + 0) * N, N)])

    @pl.when(s == 1)
    def _():
        pltpu.sync_copy(tB, bounce_v)
        pltpu.sync_copy(bounce_v, deg_out.at[pl.ds((c * 4 + 1) * N, N)])

    @pl.when(s == 2)
    def _():
        pltpu.sync_copy(tC, bounce_v)
        pltpu.sync_copy(bounce_v, deg_out.at[pl.ds((c * 4 + 2) * N, N)])

    @pl.when(s == 3)
    def _():
        pltpu.sync_copy(tD, bounce_v)
        pltpu.sync_copy(bounce_v, deg_out.at[pl.ds((c * 4 + 3) * N, N)])


_BN = 2000
_NB = N // _BN


def _proj_body(x_ref, w1_ref, rout_ref, out_ref):
    xb = x_ref[...]
    for r in range(R):
        y = jnp.dot(xb, w1_ref[r], preferred_element_type=jnp.float32)
        out_ref[r] = (y * rout_ref[:, r:r + 1]).astype(jnp.bfloat16)


def _proj_call(x, W1, rout_t):
    return pl.pallas_call(
        _proj_body,
        grid=(_NB,),
        in_specs=[
            pl.BlockSpec((_BN, IN_FEATS), lambda i: (i, 0)),
            pl.BlockSpec((R, IN_FEATS, HID), lambda i: (0, 0, 0)),
            pl.BlockSpec((_BN, R), lambda i: (i, 0)),
        ],
        out_specs=pl.BlockSpec((R, _BN, HID), lambda i: (0, i, 0)),
        out_shape=jax.ShapeDtypeStruct((R, N, HID), jnp.bfloat16),
    )(x, W1, rout_t)


@functools.partial(
    pl.kernel,
    mesh=_MESH,
    compiler_params=pltpu.CompilerParams(use_tc_tiling_on_sc=False),
    out_type=(
        jax.ShapeDtypeStruct((R * N, HID), jnp.bfloat16),
        jax.ShapeDtypeStruct((R * N,), jnp.float32),
    ),
    scratch_types=[
        pltpu.VMEM_SHARED((N, HID), jnp.bfloat16),
        pltpu.VMEM_SHARED((N, HID), jnp.bfloat16),
        pltpu.VMEM_SHARED((N,), jnp.float32),
        pltpu.VMEM_SHARED((N,), jnp.float32),
        pltpu.VMEM((CHUNK,), jnp.int32),
        pltpu.VMEM((CHUNK,), jnp.int32),
        pltpu.VMEM((CHUNK,), jnp.int32),
        pltpu.VMEM((CHUNK,), jnp.int32),
        pltpu.VMEM((CHUNK, HID), jnp.bfloat16),
        pltpu.VMEM((CHUNK, HID), jnp.bfloat16),
        pltpu.VMEM((CHUNK,), jnp.float32),
        pltpu.VMEM((CHUNK,), jnp.float32),
        pltpu.VMEM((N,), jnp.float32),
        pltpu.SemaphoreType.DMA,
        pltpu.SemaphoreType.DMA,
        pltpu.SemaphoreType.DMA,
        pltpu.SemaphoreType.DMA,
        pltpu.SemaphoreType.DMA,
        pltpu.SemaphoreType.DMA,
    ],
)
def _edge_call(s0, d0, s1, d1, s2, d2, s3, d3,
               yn0, yn1, yn2, yn3, rin0, rin1, rin2, rin3,
               zeros_2d, zeros_n,
               agg_out, s_out,
               aggA, aggB, sA, sB,
               isrcA, isrcB, idstA, idstB, rowsA, rowsB, wA, wB,
               sb_v,
               semG, semG2, semSA, semSB, semWA, semWB):
    c = lax.axis_index("c")
    s = lax.axis_index("s")
    NCH = N // CHUNK
    NCHK = EPT // CHUNK

    isrc_b = (isrcA, isrcB)
    idst_b = (idstA, idstB)
    rows_b = (rowsA, rowsB)
    w_b = (wA, wB)
    semS_b = (semSA, semSB)
    semW_b = (semWA, semWB)

    @pl.when(s < 2)
    def _():
        pltpu.sync_copy(zeros_2d, rowsA)

    @pl.when((s == 2) | (s == 3))
    def _():
        pltpu.sync_copy(zeros_n, sb_v)

    @pl.when(s == 0)
    def _():
        for k in range(NCH):
            pltpu.sync_copy(rowsA, aggA.at[pl.ds(k * CHUNK, CHUNK)])

    @pl.when(s == 1)
    def _():
        for k in range(NCH):
            pltpu.sync_copy(rowsA, aggB.at[pl.ds(k * CHUNK, CHUNK)])

    @pl.when(s == 2)
    def _():
        pltpu.sync_copy(sb_v, sA)

    @pl.when(s == 3)
    def _():
        pltpu.sync_copy(sb_v, sB)

    plsc.subcore_barrier()

    def proc(src_ref, dst_ref, yn_ref, rin_ref, agg_sh, s_sh):
        pending = []
        for k in range(NCHK):
            b = k % 2
            if k >= 2:
                pending[k - 2][0].wait()
                pending[k - 2][1].wait()
            base = s * EPT + k * CHUNK
            pltpu.sync_copy(src_ref.at[pl.ds(base, CHUNK)], isrc_b[b])
            pltpu.sync_copy(dst_ref.at[pl.ds(base, CHUNK)], idst_b[b])
            gd = pltpu.async_copy(yn_ref.at[isrc_b[b]], rows_b[b], semG)
            wd = pltpu.async_copy(rin_ref.at[idst_b[b]], w_b[b], semG2)
            gd.wait()
            wd.wait()
            sc_rows = pltpu.async_copy(rows_b[b], agg_sh.at[idst_b[b]],
                                       semS_b[b], add=True)
            sc_w = pltpu.async_copy(w_b[b], s_sh.at[isrc_b[b]],
                                    semW_b[b], add=True)
            pending.append((sc_rows, sc_w))
        for k in range(max(0, NCHK - 2), NCHK):
            pending[k][0].wait()
            pending[k][1].wait()

    @pl.when(c == 0)
    def _():
        proc(s0, d0, yn0, rin0, aggA, sA)
        proc(s1, d1, yn1, rin1, aggB, sB)

    @pl.when(c == 1)
    def _():
        proc(s2, d2, yn2, rin2, aggA, sA)
        proc(s3, d3, yn3, rin3, aggB, sB)

    plsc.subcore_barrier()

    @pl.when(s == 0)
    def _():
        for k in range(NCH):
            pltpu.sync_copy(aggA.at[pl.ds(k * CHUNK, CHUNK)], rowsA)
            pltpu.sync_copy(
                rowsA, agg_out.at[pl.ds((c * 2 + 0) * N + k * CHUNK, CHUNK)])

    @pl.when(s == 1)
    def _():
        for k in range(NCH):
            pltpu.sync_copy(aggB.at[pl.ds(k * CHUNK, CHUNK)], rowsA)
            pltpu.sync_copy(
                rowsA, agg_out.at[pl.ds((c * 2 + 1) * N + k * CHUNK, CHUNK)])

    @pl.when(s == 2)
    def _():
        pltpu.sync_copy(sA, sb_v)
        pltpu.sync_copy(sb_v, s_out.at[pl.ds((c * 2 + 0) * N, N)])

    @pl.when(s == 3)
    def _():
        pltpu.sync_copy(sB, sb_v)
        pltpu.sync_copy(sb_v, s_out.at[pl.ds((c * 2 + 1) * N, N)])


def _pool_body(agg_ref, sp_ref, rin_ref, rout_ref, b1_ref, w2_ref, b2_ref,
               wc_ref, bc_ref, out_ref, v_acc):
    i = pl.program_id(0)

    @pl.when(i == 0)
    def _():
        v_acc[...] = jnp.zeros((R, HID), jnp.float32)

    agg = agg_ref[...].astype(jnp.float32)
    hsum = (agg[0] * rin_ref[:, 0:1] + agg[1] * rin_ref[:, 1:2]
            + agg[2] * rin_ref[:, 2:3] + agg[3] * rin_ref[:, 3:4])
    b1s = jnp.sum(b1_ref[...], axis=0, keepdims=True)
    h = jnp.maximum(hsum + b1s, 0.0)
    cmat = rout_ref[...] * sp_ref[...]
    v_acc[...] += lax.dot_general(cmat, h, (((0,), (0,)), ((), ())),
                                  preferred_element_type=jnp.float32)

    @pl.when(i == _NB - 1)
    def _():
        V = v_acc[...]
        vw = lax.dot_general(V, w2_ref[...], (((1,), (1,)), ((0,), (0,))),
                             preferred_element_type=jnp.float32)
        ge = (jnp.sum(vw, axis=0, keepdims=True) * (1.0 / N)
              + jnp.sum(b2_ref[...], axis=0, keepdims=True))
        out_ref[...] = (jnp.dot(ge, wc_ref[...],
                                preferred_element_type=jnp.float32)
                        + bc_ref[...])


def _pool_call(agg, sp_t, rin_t, rout_t, b1, W2, b2, Wc, bc2):
    return pl.pallas_call(
        _pool_body,
        grid=(_NB,),
        in_specs=[
            pl.BlockSpec((R, _BN, HID), lambda i: (0, i, 0)),
            pl.BlockSpec((_BN, R), lambda i: (i, 0)),
            pl.BlockSpec((_BN, R), lambda i: (i, 0)),
            pl.BlockSpec((_BN, R), lambda i: (i, 0)),
            pl.BlockSpec((R, HID), lambda i: (0, 0)),
            pl.BlockSpec((R, HID, HID), lambda i: (0, 0, 0)),
            pl.BlockSpec((R, HID), lambda i: (0, 0)),
            pl.BlockSpec((HID, OUT), lambda i: (0, 0)),
            pl.BlockSpec((1, OUT), lambda i: (0, 0)),
        ],
        out_specs=pl.BlockSpec((1, OUT), lambda i: (0, 0)),
        out_shape=jax.ShapeDtypeStruct((1, OUT), jnp.float32),
        scratch_shapes=[pltpu.VMEM((R, HID), jnp.float32)],
    )(agg, sp_t, rin_t, rout_t, b1, W2, b2, Wc, bc2)


@jax.jit
def kernel(x, edge_index, W1, b1, W2, b2, Wc, bc):
    srcs = [edge_index[r, 0] for r in range(R)]
    dsts = [edge_index[r, 1] for r in range(R)]
    zeros_n = jnp.zeros((N,), jnp.float32)
    zeros_2d = jnp.zeros((CHUNK, HID), jnp.bfloat16)
    ones_e = jnp.ones((EPT,), jnp.float32)

    deg_flat = _deg_call(srcs[0], dsts[0], srcs[1], dsts[1],
                         srcs[2], dsts[2], srcs[3], dsts[3],
                         zeros_n, ones_e)
    deg = deg_flat.reshape(R, 2, N)
    rout = lax.rsqrt(jnp.maximum(deg[:, 0], 1.0))
    rin = lax.rsqrt(jnp.maximum(deg[:, 1], 1.0))
    rout_t = rout.T
    rin_t = rin.T

    yn = _proj_call(x, W1, rout_t)

    agg_f, s_f = _edge_call(
        srcs[0], dsts[0], srcs[1], dsts[1], srcs[2], dsts[2], srcs[3], dsts[3],
        yn[0], yn[1], yn[2], yn[3],
        rin[0], rin[1], rin[2], rin[3],
        zeros_2d, zeros_n)

    logits = _pool_call(agg_f.reshape(R, N, HID), s_f.reshape(R, N).T,
                        rin_t, rout_t, b1, W2, b2, Wc, bc.reshape(1, OUT))
    return logits.reshape(OUT)
